# baseline (speedup 1.0000x reference)
import functools

import numpy as np
import jax
import jax.numpy as jnp
from jax import lax
from jax.experimental import pallas as pl
from jax.experimental.pallas import tpu as pltpu
from jax.experimental.pallas import tpu_sc as plsc

INPUT_DIM = 3
NUM_LEVELS = 16
BASE_RES = 16
LOG2_HASHMAP = 19
DESIRED_RES = 2048
B_PTS = 262144
OUT_DIM = 2 * NUM_LEVELS
HASH_MASK = (1 << LOG2_HASHMAP) - 1
HP1 = int(np.uint32(2654435761).astype(np.int32))
HP2 = 805459861

_PLS = float(np.exp2(np.log2(DESIRED_RES / BASE_RES) / (NUM_LEVELS - 1)))
_S = float(np.log2(_PLS))


def _level_constants():
    levels = []
    off = 0
    maxp = 2 ** LOG2_HASHMAP
    for i in range(NUM_LEVELS):
        res = int(np.ceil(BASE_RES * _PLS ** i))
        params = min(maxp, (res + 1) ** INPUT_DIM)
        params = int(np.ceil(params / 8) * 8)
        scale = float(np.exp2(i * _S) * BASE_RES - 1.0)
        resolution = int(np.ceil(scale)) + 1
        stride_base = resolution + 1
        hashed = stride_base ** INPUT_DIM > params
        levels.append(dict(scale=scale, R=stride_base, hashed=hashed, offset=off))
        off += params
    return levels, off


LEVELS, TOTAL_TABLE = _level_constants()

_info = plsc.get_sparse_core_info()
NC = _info.num_cores
NS = _info.num_subcores
L = _info.num_lanes
NW = NC * NS
PPW = B_PTS // NW
C = 128
NCHUNK = PPW // C
NG = C // 16
CPF = 16
FLUSH = C * CPF
NFLUSH = PPW // FLUSH


def _phase1(xs, idx_v, w_v, cbase, lvl):
    scale = jnp.float32(lvl["scale"])
    half = jnp.float32(0.5)
    one = jnp.float32(1.0)

    def body(gi, carry):
        sl = pl.ds(cbase + gi * 16, 16)
        gsl = pl.ds(gi * 16, 16)
        g = []
        f = []
        for d in range(3):
            x = xs[d][sl]
            xn = (x + one) * half
            pos = xn * scale + half
            gg = pos.astype(jnp.int32)
            g.append(gg)
            f.append(pos - gg.astype(jnp.float32))
        t0 = [one - f[0], f[0]]
        t1 = [one - f[1], f[1]]
        t2 = [one - f[2], f[2]]
        wxy = [[t0[a] * t1[b] for b in range(2)] for a in range(2)]
        if lvl["hashed"]:
            h1 = [g[1] * jnp.int32(HP1)]
            h1.append(h1[0] + jnp.int32(HP1))
            h2 = [g[2] * jnp.int32(HP2)]
            h2.append(h2[0] + jnp.int32(HP2))
            g0 = [g[0], g[0] + jnp.int32(1)]
            off2_c = jnp.int32(2 * lvl["offset"])
            for c in range(8):
                b0, b1, b2 = c & 1, (c >> 1) & 1, (c >> 2) & 1
                idx = ((g0[b0] ^ h1[b1]) ^ h2[b2]) & jnp.int32(HASH_MASK)
                idx_v[pl.ds(c * C + gi * 16, 16)] = (idx + idx) + off2_c
                w_v[pl.ds(c * C + gi * 16, 16)] = wxy[b0][b1] * t2[b2]
        else:
            R = lvl["R"]
            s1 = [g[1] * jnp.int32(2 * R)]
            s1.append(s1[0] + jnp.int32(2 * R))
            s2 = [g[2] * jnp.int32(2 * R * R)]
            s2.append(s2[0] + jnp.int32(2 * R * R))
            g0 = [(g[0] + g[0]) + jnp.int32(2 * lvl["offset"])]
            g0.append(g0[0] + jnp.int32(2))
            for c in range(8):
                b0, b1, b2 = c & 1, (c >> 1) & 1, (c >> 2) & 1
                idx_v[pl.ds(c * C + gi * 16, 16)] = (g0[b0] + s1[b1]) + s2[b2]
                w_v[pl.ds(c * C + gi * 16, 16)] = wxy[b0][b1] * t2[b2]
        return carry

    lax.fori_loop(0, NG, body, 0)


def _phase3(f0_v, f1_v, w_v, out_v, obase, li):
    def body(gi, carry):
        acc0 = None
        acc1 = None
        for c in range(8):
            w = w_v[pl.ds(c * C + gi * 16, 16)]
            f0 = f0_v[pl.ds(c * C + gi * 16, 16)]
            f1 = f1_v[pl.ds(c * C + gi * 16, 16)]
            if c == 0:
                acc0 = w * f0
                acc1 = w * f1
            else:
                acc0 = acc0 + w * f0
                acc1 = acc1 + w * f1
        out_v[pl.ds((2 * li) * FLUSH + obase + gi * 16, 16)] = acc0
        out_v[pl.ds((2 * li + 1) * FLUSH + obase + gi * 16, 16)] = acc1
        return carry

    lax.fori_loop(0, NG, body, 0)


@functools.partial(
    pl.kernel,
    mesh=plsc.VectorSubcoreMesh(core_axis_name="c", subcore_axis_name="s"),
    out_type=jax.ShapeDtypeStruct((OUT_DIM * B_PTS,), jnp.float32),
    scratch_types=[
        pltpu.VMEM((PPW,), jnp.float32),
        pltpu.VMEM((PPW,), jnp.float32),
        pltpu.VMEM((PPW,), jnp.float32),
        pltpu.VMEM((8 * C,), jnp.int32),
        pltpu.VMEM((8 * C,), jnp.int32),
        pltpu.VMEM((8 * C,), jnp.float32),
        pltpu.VMEM((8 * C,), jnp.float32),
        pltpu.VMEM((8 * C,), jnp.float32),
        pltpu.VMEM((8 * C,), jnp.float32),
        pltpu.VMEM((8 * C,), jnp.float32),
        pltpu.VMEM((8 * C,), jnp.float32),
        pltpu.VMEM((OUT_DIM * FLUSH,), jnp.float32),
        pltpu.SemaphoreType.DMA,
        pltpu.SemaphoreType.DMA,
    ],
)
def _encode(x_hbm, t0_hbm, t1_hbm, out_hbm, x0_v, x1_v, x2_v,
            idx_a, idx_b, w_a, w_b, f0_a, f0_b, f1_a, f1_b, out_v, sem_a, sem_b):
    wid = lax.axis_index("s") * NC + lax.axis_index("c")
    base = wid * PPW
    xs = (x0_v, x1_v, x2_v)
    bufs = ((idx_a, w_a, f0_a, f1_a, sem_a), (idx_b, w_b, f0_b, f1_b, sem_b))
    for d in range(3):
        pltpu.sync_copy(x_hbm.at[pl.ds(d * B_PTS + base, PPW)], xs[d])

    def _fire(buf):
        idx_v, _, f0_v, f1_v, sem = buf
        copies = []
        for c in range(8):
            isl = idx_v.at[pl.ds(c * C, C)]
            dsl = pl.ds(c * C, C)
            copies.append(pltpu.async_copy(t0_hbm.at[isl], f0_v.at[dsl], sem))
            copies.append(pltpu.async_copy(t1_hbm.at[isl], f1_v.at[dsl], sem))
        return copies

    def flush_grp(fi, carry):
        def chunk(ici, carry2):
            cbase = fi * FLUSH + ici * C
            obase = ici * C
            _phase1(xs, bufs[0][0], bufs[0][1], cbase, LEVELS[0])
            pending = _fire(bufs[0])
            for li in range(1, NUM_LEVELS):
                b = bufs[li & 1]
                pb = bufs[(li - 1) & 1]
                _phase1(xs, b[0], b[1], cbase, LEVELS[li])
                nxt = _fire(b)
                for cp in pending:
                    cp.wait()
                _phase3(pb[2], pb[3], pb[1], out_v, obase, li - 1)
                pending = nxt
            for cp in pending:
                cp.wait()
            lb = bufs[(NUM_LEVELS - 1) & 1]
            _phase3(lb[2], lb[3], lb[1], out_v, obase, NUM_LEVELS - 1)
            return carry2

        lax.fori_loop(0, CPF, chunk, 0)
        for p in range(OUT_DIM):
            pltpu.sync_copy(
                out_v.at[pl.ds(p * FLUSH, FLUSH)],
                out_hbm.at[pl.ds(p * B_PTS + base + fi * FLUSH, FLUSH)],
            )
        return carry

    lax.fori_loop(0, NFLUSH, flush_grp, 0)


def kernel(inputs, embeddings, own_embeddings):
    table = jnp.concatenate([embeddings, own_embeddings], axis=0)
    flat = table.reshape(-1)
    flat_sh = jnp.concatenate([flat[1:], jnp.zeros((1,), jnp.float32)])
    xT = inputs.T.reshape(-1)
    out = _encode(xT, flat, flat_sh)
    return out.reshape(OUT_DIM, B_PTS).T


# speedup vs baseline: 2.5989x; 2.5989x over previous
import functools

import numpy as np
import jax
import jax.numpy as jnp
from jax import lax
from jax.experimental import pallas as pl
from jax.experimental.pallas import tpu as pltpu
from jax.experimental.pallas import tpu_sc as plsc

INPUT_DIM = 3
NUM_LEVELS = 16
BASE_RES = 16
LOG2_HASHMAP = 19
DESIRED_RES = 2048
B_PTS = 262144
OUT_DIM = 2 * NUM_LEVELS
HASH_MASK = (1 << LOG2_HASHMAP) - 1
HP1 = int(np.uint32(2654435761).astype(np.int32))
HP2 = 805459861

_PLS = float(np.exp2(np.log2(DESIRED_RES / BASE_RES) / (NUM_LEVELS - 1)))
_S = float(np.log2(_PLS))


def _level_constants():
    levels = []
    off = 0
    maxp = 2 ** LOG2_HASHMAP
    for i in range(NUM_LEVELS):
        res = int(np.ceil(BASE_RES * _PLS ** i))
        params = min(maxp, (res + 1) ** INPUT_DIM)
        params = int(np.ceil(params / 8) * 8)
        scale = float(np.exp2(i * _S) * BASE_RES - 1.0)
        resolution = int(np.ceil(scale)) + 1
        stride_base = resolution + 1
        hashed = stride_base ** INPUT_DIM > params
        levels.append(dict(scale=scale, R=stride_base, hashed=hashed, offset=off))
        off += params
    return levels, off


LEVELS, TOTAL_TABLE = _level_constants()

_info = plsc.get_sparse_core_info()
NC = _info.num_cores
NS = _info.num_subcores
L = _info.num_lanes
NW = NC * NS
PPW = B_PTS // NW
C = 512
NCHUNK = PPW // C
NG = C // 16
CPF = 2
FLUSH = C * CPF
NFLUSH = PPW // FLUSH


def _phase1(xs, idx_v, w_v, cbase, lvl):
    scale = jnp.float32(lvl["scale"])
    half = jnp.float32(0.5)
    one = jnp.float32(1.0)

    def body(gi, carry):
        sl = pl.ds(cbase + gi * 16, 16)
        gsl = pl.ds(gi * 16, 16)
        g = []
        f = []
        for d in range(3):
            x = xs[d][sl]
            xn = (x + one) * half
            pos = xn * scale + half
            gg = pos.astype(jnp.int32)
            g.append(gg)
            f.append(pos - gg.astype(jnp.float32))
        t0 = [one - f[0], f[0]]
        t1 = [one - f[1], f[1]]
        t2 = [one - f[2], f[2]]
        wxy = [[t0[a] * t1[b] for b in range(2)] for a in range(2)]
        if lvl["hashed"]:
            h1 = [g[1] * jnp.int32(HP1)]
            h1.append(h1[0] + jnp.int32(HP1))
            h2 = [g[2] * jnp.int32(HP2)]
            h2.append(h2[0] + jnp.int32(HP2))
            g0 = [g[0], g[0] + jnp.int32(1)]
            off_c = jnp.int32(lvl["offset"])
            for c in range(8):
                b0, b1, b2 = c & 1, (c >> 1) & 1, (c >> 2) & 1
                idx = ((g0[b0] ^ h1[b1]) ^ h2[b2]) & jnp.int32(HASH_MASK)
                idx_v[pl.ds(c * C + gi * 16, 16)] = idx + off_c
                w_v[pl.ds(c * C + gi * 16, 16)] = wxy[b0][b1] * t2[b2]
        else:
            R = lvl["R"]
            s1 = [g[1] * jnp.int32(R)]
            s1.append(s1[0] + jnp.int32(R))
            s2 = [g[2] * jnp.int32(R * R)]
            s2.append(s2[0] + jnp.int32(R * R))
            g0 = [g[0] + jnp.int32(lvl["offset"])]
            g0.append(g0[0] + jnp.int32(1))
            for c in range(8):
                b0, b1, b2 = c & 1, (c >> 1) & 1, (c >> 2) & 1
                idx_v[pl.ds(c * C + gi * 16, 16)] = (g0[b0] + s1[b1]) + s2[b2]
                w_v[pl.ds(c * C + gi * 16, 16)] = wxy[b0][b1] * t2[b2]
        return carry

    lax.fori_loop(0, NG, body, 0)


def _phase3(f0_v, f1_v, w_v, out_v, obase, li):
    def body(gi, carry):
        acc0 = None
        acc1 = None
        for c in range(8):
            w = w_v[pl.ds(c * C + gi * 16, 16)]
            f0 = f0_v[pl.ds(c * C + gi * 16, 16)]
            f1 = f1_v[pl.ds(c * C + gi * 16, 16)]
            if c == 0:
                acc0 = w * f0
                acc1 = w * f1
            else:
                acc0 = acc0 + w * f0
                acc1 = acc1 + w * f1
        out_v[pl.ds((2 * li) * FLUSH + obase + gi * 16, 16)] = acc0
        out_v[pl.ds((2 * li + 1) * FLUSH + obase + gi * 16, 16)] = acc1
        return carry

    lax.fori_loop(0, NG, body, 0)


@functools.partial(
    pl.kernel,
    mesh=plsc.VectorSubcoreMesh(core_axis_name="c", subcore_axis_name="s"),
    out_type=jax.ShapeDtypeStruct((OUT_DIM * B_PTS,), jnp.float32),
    scratch_types=[
        pltpu.VMEM((FLUSH,), jnp.float32),
        pltpu.VMEM((FLUSH,), jnp.float32),
        pltpu.VMEM((FLUSH,), jnp.float32),
        pltpu.VMEM((8 * C,), jnp.int32),
        pltpu.VMEM((8 * C,), jnp.int32),
        pltpu.VMEM((8 * C,), jnp.float32),
        pltpu.VMEM((8 * C,), jnp.float32),
        pltpu.VMEM((8 * C,), jnp.float32),
        pltpu.VMEM((8 * C,), jnp.float32),
        pltpu.VMEM((8 * C,), jnp.float32),
        pltpu.VMEM((8 * C,), jnp.float32),
        pltpu.VMEM((OUT_DIM * FLUSH,), jnp.float32),
        pltpu.SemaphoreType.DMA,
        pltpu.SemaphoreType.DMA,
    ],
)
def _encode(x_hbm, t0_hbm, t1_hbm, out_hbm, x0_v, x1_v, x2_v,
            idx_a, idx_b, w_a, w_b, f0_a, f0_b, f1_a, f1_b, out_v, sem_a, sem_b):
    wid = lax.axis_index("s") * NC + lax.axis_index("c")
    base = wid * PPW
    xs = (x0_v, x1_v, x2_v)
    bufs = ((idx_a, w_a, f0_a, f1_a, sem_a), (idx_b, w_b, f0_b, f1_b, sem_b))

    def _fire(buf):
        idx_v, _, f0_v, f1_v, sem = buf
        copies = []
        for c in range(8):
            isl = idx_v.at[pl.ds(c * C, C)]
            dsl = pl.ds(c * C, C)
            copies.append(pltpu.async_copy(t0_hbm.at[isl], f0_v.at[dsl], sem))
            copies.append(pltpu.async_copy(t1_hbm.at[isl], f1_v.at[dsl], sem))
        return copies

    def flush_grp(fi, carry):
        for d in range(3):
            pltpu.sync_copy(
                x_hbm.at[pl.ds(d * B_PTS + base + fi * FLUSH, FLUSH)], xs[d]
            )

        def chunk(ici, carry2):
            cbase = ici * C
            obase = ici * C
            _phase1(xs, bufs[0][0], bufs[0][1], cbase, LEVELS[0])
            pending = _fire(bufs[0])
            for li in range(1, NUM_LEVELS):
                b = bufs[li & 1]
                pb = bufs[(li - 1) & 1]
                _phase1(xs, b[0], b[1], cbase, LEVELS[li])
                nxt = _fire(b)
                for cp in pending:
                    cp.wait()
                _phase3(pb[2], pb[3], pb[1], out_v, obase, li - 1)
                pending = nxt
            for cp in pending:
                cp.wait()
            lb = bufs[(NUM_LEVELS - 1) & 1]
            _phase3(lb[2], lb[3], lb[1], out_v, obase, NUM_LEVELS - 1)
            return carry2

        lax.fori_loop(0, CPF, chunk, 0)
        for p in range(OUT_DIM):
            pltpu.sync_copy(
                out_v.at[pl.ds(p * FLUSH, FLUSH)],
                out_hbm.at[pl.ds(p * B_PTS + base + fi * FLUSH, FLUSH)],
            )
        return carry

    lax.fori_loop(0, NFLUSH, flush_grp, 0)


def kernel(inputs, embeddings, own_embeddings):
    table = jnp.concatenate([embeddings, own_embeddings], axis=0)
    xT = inputs.T.reshape(-1)
    out = _encode(xT, table[:, 0], table[:, 1])
    return out.reshape(OUT_DIM, B_PTS).T


# speedup vs baseline: 2.6121x; 1.0051x over previous
import functools

import numpy as np
import jax
import jax.numpy as jnp
from jax import lax
from jax.experimental import pallas as pl
from jax.experimental.pallas import tpu as pltpu
from jax.experimental.pallas import tpu_sc as plsc

INPUT_DIM = 3
NUM_LEVELS = 16
BASE_RES = 16
LOG2_HASHMAP = 19
DESIRED_RES = 2048
B_PTS = 262144
OUT_DIM = 2 * NUM_LEVELS
HASH_MASK = (1 << LOG2_HASHMAP) - 1
HP1 = int(np.uint32(2654435761).astype(np.int32))
HP2 = 805459861

_PLS = float(np.exp2(np.log2(DESIRED_RES / BASE_RES) / (NUM_LEVELS - 1)))
_S = float(np.log2(_PLS))


def _level_constants():
    levels = []
    off = 0
    maxp = 2 ** LOG2_HASHMAP
    for i in range(NUM_LEVELS):
        res = int(np.ceil(BASE_RES * _PLS ** i))
        params = min(maxp, (res + 1) ** INPUT_DIM)
        params = int(np.ceil(params / 8) * 8)
        scale = float(np.exp2(i * _S) * BASE_RES - 1.0)
        resolution = int(np.ceil(scale)) + 1
        stride_base = resolution + 1
        hashed = stride_base ** INPUT_DIM > params
        levels.append(dict(scale=scale, R=stride_base, hashed=hashed, offset=off))
        off += params
    return levels, off


LEVELS, TOTAL_TABLE = _level_constants()

_info = plsc.get_sparse_core_info()
NC = _info.num_cores
NS = _info.num_subcores
L = _info.num_lanes
NW = NC * NS
PPW = B_PTS // NW
C = 512
NCHUNK = PPW // C
NG = C // 16
CPF = 2
FLUSH = C * CPF
NFLUSH = PPW // FLUSH


def _phase1(xs, idx_v, cbase, lvl):
    scale = jnp.float32(lvl["scale"])
    half = jnp.float32(0.5)
    one = jnp.float32(1.0)

    def body(gi, carry):
        sl = pl.ds(cbase + gi * 16, 16)
        g = []
        for d in range(3):
            x = xs[d][sl]
            xn = (x + one) * half
            pos = xn * scale + half
            g.append(pos.astype(jnp.int32))
        if lvl["hashed"]:
            h1 = [g[1] * jnp.int32(HP1)]
            h1.append(h1[0] + jnp.int32(HP1))
            h2 = [g[2] * jnp.int32(HP2)]
            h2.append(h2[0] + jnp.int32(HP2))
            g0 = [g[0], g[0] + jnp.int32(1)]
            off_c = jnp.int32(lvl["offset"])
            for c in range(8):
                b0, b1, b2 = c & 1, (c >> 1) & 1, (c >> 2) & 1
                idx = ((g0[b0] ^ h1[b1]) ^ h2[b2]) & jnp.int32(HASH_MASK)
                idx_v[pl.ds(c * C + gi * 16, 16)] = idx + off_c
        else:
            R = lvl["R"]
            s1 = [g[1] * jnp.int32(R)]
            s1.append(s1[0] + jnp.int32(R))
            s2 = [g[2] * jnp.int32(R * R)]
            s2.append(s2[0] + jnp.int32(R * R))
            g0 = [g[0] + jnp.int32(lvl["offset"])]
            g0.append(g0[0] + jnp.int32(1))
            for c in range(8):
                b0, b1, b2 = c & 1, (c >> 1) & 1, (c >> 2) & 1
                idx_v[pl.ds(c * C + gi * 16, 16)] = (g0[b0] + s1[b1]) + s2[b2]
        return carry

    lax.fori_loop(0, NG, body, 0)


def _phase3(xs, f0_v, f1_v, out_v, cbase, obase, li, lvl):
    scale = jnp.float32(lvl["scale"])
    half = jnp.float32(0.5)
    one = jnp.float32(1.0)

    def body(gi, carry):
        sl = pl.ds(cbase + gi * 16, 16)
        f = []
        for d in range(3):
            x = xs[d][sl]
            xn = (x + one) * half
            pos = xn * scale + half
            gg = pos.astype(jnp.int32)
            f.append(pos - gg.astype(jnp.float32))
        t0 = [one - f[0], f[0]]
        t1 = [one - f[1], f[1]]
        t2 = [one - f[2], f[2]]
        wxy = [[t0[a] * t1[b] for b in range(2)] for a in range(2)]
        acc0 = None
        acc1 = None
        for c in range(8):
            b0, b1, b2 = c & 1, (c >> 1) & 1, (c >> 2) & 1
            w = wxy[b0][b1] * t2[b2]
            f0 = f0_v[pl.ds(c * C + gi * 16, 16)]
            f1 = f1_v[pl.ds(c * C + gi * 16, 16)]
            if c == 0:
                acc0 = w * f0
                acc1 = w * f1
            else:
                acc0 = acc0 + w * f0
                acc1 = acc1 + w * f1
        out_v[pl.ds((2 * li) * FLUSH + obase + gi * 16, 16)] = acc0
        out_v[pl.ds((2 * li + 1) * FLUSH + obase + gi * 16, 16)] = acc1
        return carry

    lax.fori_loop(0, NG, body, 0)


@functools.partial(
    pl.kernel,
    mesh=plsc.VectorSubcoreMesh(core_axis_name="c", subcore_axis_name="s"),
    out_type=jax.ShapeDtypeStruct((OUT_DIM * B_PTS,), jnp.float32),
    scratch_types=[
        pltpu.VMEM((FLUSH,), jnp.float32),
        pltpu.VMEM((FLUSH,), jnp.float32),
        pltpu.VMEM((FLUSH,), jnp.float32),
        pltpu.VMEM((8 * C,), jnp.int32),
        pltpu.VMEM((8 * C,), jnp.int32),
        pltpu.VMEM((8 * C,), jnp.float32),
        pltpu.VMEM((8 * C,), jnp.float32),
        pltpu.VMEM((8 * C,), jnp.float32),
        pltpu.VMEM((8 * C,), jnp.float32),
        pltpu.VMEM((8 * C,), jnp.float32),
        pltpu.VMEM((8 * C,), jnp.float32),
        pltpu.VMEM((OUT_DIM * FLUSH,), jnp.float32),
        pltpu.SemaphoreType.DMA,
        pltpu.SemaphoreType.DMA,
    ],
)
def _encode(x_hbm, t0_hbm, t1_hbm, out_hbm, x0_v, x1_v, x2_v,
            idx_a, idx_b, w_a, w_b, f0_a, f0_b, f1_a, f1_b, out_v, sem_a, sem_b):
    wid = lax.axis_index("s") * NC + lax.axis_index("c")
    base = wid * PPW
    xs = (x0_v, x1_v, x2_v)
    bufs = ((idx_a, w_a, f0_a, f1_a, sem_a), (idx_b, w_b, f0_b, f1_b, sem_b))

    def _fire(buf):
        idx_v, _, f0_v, f1_v, sem = buf
        copies = []
        for c in range(8):
            isl = idx_v.at[pl.ds(c * C, C)]
            dsl = pl.ds(c * C, C)
            copies.append(pltpu.async_copy(t0_hbm.at[isl], f0_v.at[dsl], sem))
            copies.append(pltpu.async_copy(t1_hbm.at[isl], f1_v.at[dsl], sem))
        return copies

    def flush_grp(fi, carry):
        for d in range(3):
            pltpu.sync_copy(
                x_hbm.at[pl.ds(d * B_PTS + base + fi * FLUSH, FLUSH)], xs[d]
            )

        def chunk(ici, carry2):
            cbase = ici * C
            obase = ici * C
            _phase1(xs, bufs[0][0], cbase, LEVELS[0])
            pending = _fire(bufs[0])
            for li in range(1, NUM_LEVELS):
                b = bufs[li & 1]
                pb = bufs[(li - 1) & 1]
                _phase1(xs, b[0], cbase, LEVELS[li])
                nxt = _fire(b)
                for cp in pending:
                    cp.wait()
                _phase3(xs, pb[2], pb[3], out_v, cbase, obase, li - 1, LEVELS[li - 1])
                pending = nxt
            for cp in pending:
                cp.wait()
            lb = bufs[(NUM_LEVELS - 1) & 1]
            _phase3(xs, lb[2], lb[3], out_v, cbase, obase, NUM_LEVELS - 1, LEVELS[NUM_LEVELS - 1])
            return carry2

        lax.fori_loop(0, CPF, chunk, 0)
        for p in range(OUT_DIM):
            pltpu.sync_copy(
                out_v.at[pl.ds(p * FLUSH, FLUSH)],
                out_hbm.at[pl.ds(p * B_PTS + base + fi * FLUSH, FLUSH)],
            )
        return carry

    lax.fori_loop(0, NFLUSH, flush_grp, 0)


def kernel(inputs, embeddings, own_embeddings):
    table = jnp.concatenate([embeddings, own_embeddings], axis=0)
    xT = inputs.T.reshape(-1)
    out = _encode(xT, table[:, 0], table[:, 1])
    return out.reshape(OUT_DIM, B_PTS).T


# speedup vs baseline: 2.6377x; 1.0098x over previous
import functools

import numpy as np
import jax
import jax.numpy as jnp
from jax import lax
from jax.experimental import pallas as pl
from jax.experimental.pallas import tpu as pltpu
from jax.experimental.pallas import tpu_sc as plsc

INPUT_DIM = 3
NUM_LEVELS = 16
BASE_RES = 16
LOG2_HASHMAP = 19
DESIRED_RES = 2048
B_PTS = 262144
OUT_DIM = 2 * NUM_LEVELS
HASH_MASK = (1 << LOG2_HASHMAP) - 1
HP1 = int(np.uint32(2654435761).astype(np.int32))
HP2 = 805459861

_PLS = float(np.exp2(np.log2(DESIRED_RES / BASE_RES) / (NUM_LEVELS - 1)))
_S = float(np.log2(_PLS))


def _level_constants():
    levels = []
    off = 0
    maxp = 2 ** LOG2_HASHMAP
    for i in range(NUM_LEVELS):
        res = int(np.ceil(BASE_RES * _PLS ** i))
        params = min(maxp, (res + 1) ** INPUT_DIM)
        params = int(np.ceil(params / 8) * 8)
        scale = float(np.exp2(i * _S) * BASE_RES - 1.0)
        resolution = int(np.ceil(scale)) + 1
        stride_base = resolution + 1
        hashed = stride_base ** INPUT_DIM > params
        levels.append(dict(scale=scale, R=stride_base, hashed=hashed, offset=off))
        off += params
    return levels, off


LEVELS, TOTAL_TABLE = _level_constants()

_info = plsc.get_sparse_core_info()
NC = _info.num_cores
NS = _info.num_subcores
L = _info.num_lanes
NW = NC * NS
PPW = B_PTS // NW
C = 128
NCHUNK = PPW // C
NG = C // 16
CPF = 16
FLUSH = C * CPF
NFLUSH = PPW // FLUSH


def _phase1(xs, idx_v, w_v, cbase, lvl):
    scale = jnp.float32(lvl["scale"])
    half = jnp.float32(0.5)
    one = jnp.float32(1.0)

    def body(gi, carry):
        sl = pl.ds(cbase + gi * 16, 16)
        gsl = pl.ds(gi * 16, 16)
        g = []
        f = []
        for d in range(3):
            x = xs[d][sl]
            xn = (x + one) * half
            pos = xn * scale + half
            gg = pos.astype(jnp.int32)
            g.append(gg)
            f.append(pos - gg.astype(jnp.float32))
        t0 = [one - f[0], f[0]]
        t1 = [one - f[1], f[1]]
        t2 = [one - f[2], f[2]]
        wxy = [[t0[a] * t1[b] for b in range(2)] for a in range(2)]
        if lvl["hashed"]:
            h1 = [g[1] * jnp.int32(HP1)]
            h1.append(h1[0] + jnp.int32(HP1))
            h2 = [g[2] * jnp.int32(HP2)]
            h2.append(h2[0] + jnp.int32(HP2))
            g0 = [g[0], g[0] + jnp.int32(1)]
            off_c = jnp.int32(lvl["offset"])
            for c in range(8):
                b0, b1, b2 = c & 1, (c >> 1) & 1, (c >> 2) & 1
                idx = ((g0[b0] ^ h1[b1]) ^ h2[b2]) & jnp.int32(HASH_MASK)
                idx_v[pl.ds(c * C + gi * 16, 16)] = idx + off_c
                w_v[pl.ds(c * C + gi * 16, 16)] = wxy[b0][b1] * t2[b2]
        else:
            R = lvl["R"]
            s1 = [g[1] * jnp.int32(R)]
            s1.append(s1[0] + jnp.int32(R))
            s2 = [g[2] * jnp.int32(R * R)]
            s2.append(s2[0] + jnp.int32(R * R))
            g0 = [g[0] + jnp.int32(lvl["offset"])]
            g0.append(g0[0] + jnp.int32(1))
            for c in range(8):
                b0, b1, b2 = c & 1, (c >> 1) & 1, (c >> 2) & 1
                idx_v[pl.ds(c * C + gi * 16, 16)] = (g0[b0] + s1[b1]) + s2[b2]
                w_v[pl.ds(c * C + gi * 16, 16)] = wxy[b0][b1] * t2[b2]
        return carry

    lax.fori_loop(0, NG, body, 0)


def _phase3(f0_v, f1_v, w_v, out_v, obase, li):
    def body(gi, carry):
        acc0 = None
        acc1 = None
        for c in range(8):
            w = w_v[pl.ds(c * C + gi * 16, 16)]
            f0 = f0_v[pl.ds(c * C + gi * 16, 16)]
            f1 = f1_v[pl.ds(c * C + gi * 16, 16)]
            if c == 0:
                acc0 = w * f0
                acc1 = w * f1
            else:
                acc0 = acc0 + w * f0
                acc1 = acc1 + w * f1
        out_v[pl.ds((2 * li) * FLUSH + obase + gi * 16, 16)] = acc0
        out_v[pl.ds((2 * li + 1) * FLUSH + obase + gi * 16, 16)] = acc1
        return carry

    lax.fori_loop(0, NG, body, 0)


@functools.partial(
    pl.kernel,
    mesh=plsc.VectorSubcoreMesh(core_axis_name="c", subcore_axis_name="s"),
    out_type=jax.ShapeDtypeStruct((OUT_DIM * B_PTS,), jnp.float32),
    scratch_types=[
        pltpu.VMEM((PPW,), jnp.float32),
        pltpu.VMEM((PPW,), jnp.float32),
        pltpu.VMEM((PPW,), jnp.float32),
        pltpu.VMEM((8 * C,), jnp.int32),
        pltpu.VMEM((8 * C,), jnp.int32),
        pltpu.VMEM((8 * C,), jnp.float32),
        pltpu.VMEM((8 * C,), jnp.float32),
        pltpu.VMEM((8 * C,), jnp.float32),
        pltpu.VMEM((8 * C,), jnp.float32),
        pltpu.VMEM((8 * C,), jnp.float32),
        pltpu.VMEM((8 * C,), jnp.float32),
        pltpu.VMEM((OUT_DIM * FLUSH,), jnp.float32),
        pltpu.SemaphoreType.DMA,
        pltpu.SemaphoreType.DMA,
    ],
)
def _encode(x_hbm, t0_hbm, t1_hbm, out_hbm, x0_v, x1_v, x2_v,
            idx_a, idx_b, w_a, w_b, f0_a, f0_b, f1_a, f1_b, out_v, sem_a, sem_b):
    wid = lax.axis_index("s") * NC + lax.axis_index("c")
    base = wid * PPW
    xs = (x0_v, x1_v, x2_v)
    bufs = ((idx_a, w_a, f0_a, f1_a, sem_a), (idx_b, w_b, f0_b, f1_b, sem_b))
    for d in range(3):
        pltpu.sync_copy(x_hbm.at[pl.ds(d * B_PTS + base, PPW)], xs[d])

    def _fire(buf):
        idx_v, _, f0_v, f1_v, sem = buf
        copies = []
        for c in range(8):
            isl = idx_v.at[pl.ds(c * C, C)]
            dsl = pl.ds(c * C, C)
            copies.append(pltpu.async_copy(t0_hbm.at[isl], f0_v.at[dsl], sem))
            copies.append(pltpu.async_copy(t1_hbm.at[isl], f1_v.at[dsl], sem))
        return copies

    def flush_grp(fi, carry):
        def chunk(ici, carry2):
            cbase = fi * FLUSH + ici * C
            obase = ici * C
            _phase1(xs, bufs[0][0], bufs[0][1], cbase, LEVELS[0])
            pending = _fire(bufs[0])
            for li in range(1, NUM_LEVELS):
                b = bufs[li & 1]
                pb = bufs[(li - 1) & 1]
                _phase1(xs, b[0], b[1], cbase, LEVELS[li])
                nxt = _fire(b)
                for cp in pending:
                    cp.wait()
                _phase3(pb[2], pb[3], pb[1], out_v, obase, li - 1)
                pending = nxt
            for cp in pending:
                cp.wait()
            lb = bufs[(NUM_LEVELS - 1) & 1]
            _phase3(lb[2], lb[3], lb[1], out_v, obase, NUM_LEVELS - 1)
            return carry2

        lax.fori_loop(0, CPF, chunk, 0)
        for p in range(OUT_DIM):
            pltpu.sync_copy(
                out_v.at[pl.ds(p * FLUSH, FLUSH)],
                out_hbm.at[pl.ds(p * B_PTS + base + fi * FLUSH, FLUSH)],
            )
        return carry

    lax.fori_loop(0, NFLUSH, flush_grp, 0)


def kernel(inputs, embeddings, own_embeddings):
    table = jnp.concatenate([embeddings, own_embeddings], axis=0)
    xT = inputs.T.reshape(-1)
    out = _encode(xT, table[:, 0], table[:, 1])
    return out.reshape(OUT_DIM, B_PTS).T


# speedup vs baseline: 2.6593x; 1.0082x over previous
import functools

import numpy as np
import jax
import jax.numpy as jnp
from jax import lax
from jax.experimental import pallas as pl
from jax.experimental.pallas import tpu as pltpu
from jax.experimental.pallas import tpu_sc as plsc

INPUT_DIM = 3
NUM_LEVELS = 16
BASE_RES = 16
LOG2_HASHMAP = 19
DESIRED_RES = 2048
B_PTS = 262144
OUT_DIM = 2 * NUM_LEVELS
HASH_MASK = (1 << LOG2_HASHMAP) - 1
HP1 = int(np.uint32(2654435761).astype(np.int32))
HP2 = 805459861

_PLS = float(np.exp2(np.log2(DESIRED_RES / BASE_RES) / (NUM_LEVELS - 1)))
_S = float(np.log2(_PLS))


def _level_constants():
    levels = []
    off = 0
    maxp = 2 ** LOG2_HASHMAP
    for i in range(NUM_LEVELS):
        res = int(np.ceil(BASE_RES * _PLS ** i))
        params = min(maxp, (res + 1) ** INPUT_DIM)
        params = int(np.ceil(params / 8) * 8)
        scale = float(np.exp2(i * _S) * BASE_RES - 1.0)
        resolution = int(np.ceil(scale)) + 1
        stride_base = resolution + 1
        hashed = stride_base ** INPUT_DIM > params
        levels.append(dict(scale=scale, R=stride_base, hashed=hashed, offset=off))
        off += params
    return levels, off


LEVELS, TOTAL_TABLE = _level_constants()

_info = plsc.get_sparse_core_info()
NC = _info.num_cores
NS = _info.num_subcores
L = _info.num_lanes
NW = NC * NS
PPW = B_PTS // NW
C = 128
NCHUNK = PPW // C
NG = C // 16
CPF = 16
FLUSH = C * CPF
NFLUSH = PPW // FLUSH


def _phase1(xs, idx_v, w_v, cbase, lvl):
    scale = jnp.float32(lvl["scale"])
    half = jnp.float32(0.5)
    one = jnp.float32(1.0)

    def body(gi, carry):
        sl = pl.ds(cbase + gi * 16, 16)
        gsl = pl.ds(gi * 16, 16)
        g = []
        f = []
        for d in range(3):
            xn = xs[d][sl]
            pos = xn * scale + half
            gg = pos.astype(jnp.int32)
            g.append(gg)
            f.append(pos - gg.astype(jnp.float32))
        t0 = [one - f[0], f[0]]
        t1 = [one - f[1], f[1]]
        t2 = [one - f[2], f[2]]
        wxy = [[t0[a] * t1[b] for b in range(2)] for a in range(2)]
        if lvl["hashed"]:
            h1 = [g[1] * jnp.int32(HP1)]
            h1.append(h1[0] + jnp.int32(HP1))
            h2 = [g[2] * jnp.int32(HP2)]
            h2.append(h2[0] + jnp.int32(HP2))
            g0 = [g[0], g[0] + jnp.int32(1)]
            off_c = jnp.int32(lvl["offset"])
            for c in range(8):
                b0, b1, b2 = c & 1, (c >> 1) & 1, (c >> 2) & 1
                idx = ((g0[b0] ^ h1[b1]) ^ h2[b2]) & jnp.int32(HASH_MASK)
                idx_v[pl.ds(c * C + gi * 16, 16)] = idx + off_c
                w_v[pl.ds(c * C + gi * 16, 16)] = wxy[b0][b1] * t2[b2]
        else:
            R = lvl["R"]
            s1 = [g[1] * jnp.int32(R)]
            s1.append(s1[0] + jnp.int32(R))
            s2 = [g[2] * jnp.int32(R * R)]
            s2.append(s2[0] + jnp.int32(R * R))
            g0 = [g[0] + jnp.int32(lvl["offset"])]
            g0.append(g0[0] + jnp.int32(1))
            for c in range(8):
                b0, b1, b2 = c & 1, (c >> 1) & 1, (c >> 2) & 1
                idx_v[pl.ds(c * C + gi * 16, 16)] = (g0[b0] + s1[b1]) + s2[b2]
                w_v[pl.ds(c * C + gi * 16, 16)] = wxy[b0][b1] * t2[b2]
        return carry

    lax.fori_loop(0, NG, body, 0)


def _phase3(f0_v, f1_v, w_v, out_v, obase, li):
    def body(gi, carry):
        acc0 = None
        acc1 = None
        for c in range(8):
            w = w_v[pl.ds(c * C + gi * 16, 16)]
            f0 = f0_v[pl.ds(c * C + gi * 16, 16)]
            f1 = f1_v[pl.ds(c * C + gi * 16, 16)]
            if c == 0:
                acc0 = w * f0
                acc1 = w * f1
            else:
                acc0 = acc0 + w * f0
                acc1 = acc1 + w * f1
        out_v[pl.ds((2 * li) * FLUSH + obase + gi * 16, 16)] = acc0
        out_v[pl.ds((2 * li + 1) * FLUSH + obase + gi * 16, 16)] = acc1
        return carry

    lax.fori_loop(0, NG, body, 0)


@functools.partial(
    pl.kernel,
    mesh=plsc.VectorSubcoreMesh(core_axis_name="c", subcore_axis_name="s"),
    out_type=jax.ShapeDtypeStruct((OUT_DIM * B_PTS,), jnp.float32),
    scratch_types=[
        pltpu.VMEM((PPW,), jnp.float32),
        pltpu.VMEM((PPW,), jnp.float32),
        pltpu.VMEM((PPW,), jnp.float32),
        pltpu.VMEM((8 * C,), jnp.int32),
        pltpu.VMEM((8 * C,), jnp.int32),
        pltpu.VMEM((8 * C,), jnp.float32),
        pltpu.VMEM((8 * C,), jnp.float32),
        pltpu.VMEM((8 * C,), jnp.float32),
        pltpu.VMEM((8 * C,), jnp.float32),
        pltpu.VMEM((8 * C,), jnp.float32),
        pltpu.VMEM((8 * C,), jnp.float32),
        pltpu.VMEM((OUT_DIM * FLUSH,), jnp.float32),
        pltpu.SemaphoreType.DMA,
        pltpu.SemaphoreType.DMA,
    ],
)
def _encode(x_hbm, t0_hbm, t1_hbm, out_hbm, x0_v, x1_v, x2_v,
            idx_a, idx_b, w_a, w_b, f0_a, f0_b, f1_a, f1_b, out_v, sem_a, sem_b):
    wid = lax.axis_index("s") * NC + lax.axis_index("c")
    base = wid * PPW
    xs = (x0_v, x1_v, x2_v)
    bufs = ((idx_a, w_a, f0_a, f1_a, sem_a), (idx_b, w_b, f0_b, f1_b, sem_b))
    for d in range(3):
        pltpu.sync_copy(x_hbm.at[pl.ds(d * B_PTS + base, PPW)], xs[d])
    half_c = jnp.float32(0.5)
    one_c = jnp.float32(1.0)

    def norm(i, carry):
        for d in range(3):
            xv = xs[d][pl.ds(i * 16, 16)]
            xs[d][pl.ds(i * 16, 16)] = (xv + one_c) * half_c
        return carry

    lax.fori_loop(0, PPW // 16, norm, 0)

    def _fire(buf):
        idx_v, _, f0_v, f1_v, sem = buf
        copies = []
        for c in range(8):
            isl = idx_v.at[pl.ds(c * C, C)]
            dsl = pl.ds(c * C, C)
            copies.append(pltpu.async_copy(t0_hbm.at[isl], f0_v.at[dsl], sem))
            copies.append(pltpu.async_copy(t1_hbm.at[isl], f1_v.at[dsl], sem))
        return copies

    def flush_grp(fi, carry):
        def chunk(ici, carry2):
            cbase = fi * FLUSH + ici * C
            obase = ici * C
            _phase1(xs, bufs[0][0], bufs[0][1], cbase, LEVELS[0])
            pending = _fire(bufs[0])
            for li in range(1, NUM_LEVELS):
                b = bufs[li & 1]
                pb = bufs[(li - 1) & 1]
                _phase1(xs, b[0], b[1], cbase, LEVELS[li])
                nxt = _fire(b)
                for cp in pending:
                    cp.wait()
                _phase3(pb[2], pb[3], pb[1], out_v, obase, li - 1)
                pending = nxt
            for cp in pending:
                cp.wait()
            lb = bufs[(NUM_LEVELS - 1) & 1]
            _phase3(lb[2], lb[3], lb[1], out_v, obase, NUM_LEVELS - 1)
            return carry2

        lax.fori_loop(0, CPF, chunk, 0)
        for p in range(OUT_DIM):
            pltpu.sync_copy(
                out_v.at[pl.ds(p * FLUSH, FLUSH)],
                out_hbm.at[pl.ds(p * B_PTS + base + fi * FLUSH, FLUSH)],
            )
        return carry

    lax.fori_loop(0, NFLUSH, flush_grp, 0)


def kernel(inputs, embeddings, own_embeddings):
    table = jnp.concatenate([embeddings, own_embeddings], axis=0)
    xT = inputs.T.reshape(-1)
    out = _encode(xT, table[:, 0], table[:, 1])
    return out.reshape(OUT_DIM, B_PTS).T
